# baseline (speedup 1.0000x reference)
import functools

import jax
import jax.numpy as jnp
from jax import lax
from jax.experimental import pallas as pl
from jax.experimental.pallas import tpu as pltpu

E = 8
NZ = 8
NEXP = E + NZ
TOPK = 2
D = 2048
DFF = 1408
T = 2048
SCALE = 2.5

BT = 256
BF = 128


def _router_weights(x, wr, bias):
    logits = lax.dot_general(x, wr, (((1,), (1,)), ((), ())),
                             preferred_element_type=jnp.float32)
    scores = jax.nn.sigmoid(logits)
    biased = scores + bias
    ii = lax.broadcasted_iota(jnp.int32, biased.shape, 1)
    big = jnp.int32(NEXP)
    m1 = jnp.max(biased, axis=1, keepdims=True)
    i1 = jnp.min(jnp.where(biased >= m1, ii, big), axis=1, keepdims=True)
    oh1 = ii == i1
    b2 = jnp.where(oh1, -jnp.inf, biased)
    m2 = jnp.max(b2, axis=1, keepdims=True)
    i2 = jnp.min(jnp.where(b2 >= m2, ii, big), axis=1, keepdims=True)
    oh2 = ii == i2
    return jnp.where(oh1 | oh2, scores, 0.0)


def _moe_body(x_ref, wr_ref, bias_ref, wg_ref, wu_ref, wd_ref, out_ref, w_ref):
    e = pl.program_id(1)
    f = pl.program_id(2)

    @pl.when((e == 0) & (f == 0))
    def _init():
        w = _router_weights(x_ref[...], wr_ref[...], bias_ref[...])
        w_ref[...] = w
        zero_w = jnp.sum(w[:, E:], axis=1, keepdims=True)
        out_ref[...] = zero_w * x_ref[...]

    x = x_ref[...]
    wg = wg_ref[0]
    wu = wu_ref[0]
    wd = wd_ref[0]
    g = lax.dot_general(x, wg, (((1,), (1,)), ((), ())),
                        preferred_element_type=jnp.float32)
    u = lax.dot_general(x, wu, (((1,), (1,)), ((), ())),
                        preferred_element_type=jnp.float32)
    h = g * jax.nn.sigmoid(g) * u
    y = lax.dot_general(h, wd, (((1,), (1,)), ((), ())),
                        preferred_element_type=jnp.float32)
    w = w_ref[...]
    col = lax.broadcasted_iota(jnp.int32, w.shape, 1)
    we = jnp.sum(jnp.where(col == e, w, 0.0), axis=1, keepdims=True)
    out_ref[...] += (SCALE * we) * y


@jax.jit
def _moe(x, wr, bias2d, wg, wu, wd):
    grid = (T // BT, E, DFF // BF)
    return pl.pallas_call(
        _moe_body,
        grid=grid,
        in_specs=[
            pl.BlockSpec((BT, D), lambda t, e, f: (t, 0)),
            pl.BlockSpec((NEXP, D), lambda t, e, f: (0, 0)),
            pl.BlockSpec((1, NEXP), lambda t, e, f: (0, 0)),
            pl.BlockSpec((1, BF, D), lambda t, e, f: (e, f, 0)),
            pl.BlockSpec((1, BF, D), lambda t, e, f: (e, f, 0)),
            pl.BlockSpec((1, D, BF), lambda t, e, f: (e, 0, f)),
        ],
        out_specs=pl.BlockSpec((BT, D), lambda t, e, f: (t, 0)),
        out_shape=jax.ShapeDtypeStruct((T, D), jnp.float32),
        scratch_shapes=[pltpu.VMEM((BT, NEXP), jnp.float32)],
        compiler_params=pltpu.CompilerParams(
            dimension_semantics=("arbitrary", "arbitrary", "arbitrary"),
        ),
    )(x, wr, bias2d, wg, wu, wd)


def kernel(hidden_states, W_router, correction_bias, W_gate, W_up, W_down):
    bias2d = correction_bias.reshape(1, NEXP)
    return _moe(hidden_states, W_router, bias2d, W_gate, W_up, W_down)


# speedup vs baseline: 1.3669x; 1.3669x over previous
import functools

import jax
import jax.numpy as jnp
from jax import lax
from jax.experimental import pallas as pl
from jax.experimental.pallas import tpu as pltpu

E = 8
NZ = 8
NEXP = E + NZ
TOPK = 2
D = 2048
DFF = 1408
T = 2048
SCALE = 2.5

BT = 256
FC = 128
DC = 512


def _router_weights(x, wr, bias):
    logits = lax.dot_general(x, wr, (((1,), (1,)), ((), ())),
                             preferred_element_type=jnp.float32)
    scores = jax.nn.sigmoid(logits)
    biased = scores + bias
    ii = lax.broadcasted_iota(jnp.int32, biased.shape, 1)
    big = jnp.int32(NEXP)
    m1 = jnp.max(biased, axis=1, keepdims=True)
    i1 = jnp.min(jnp.where(biased >= m1, ii, big), axis=1, keepdims=True)
    oh1 = ii == i1
    b2 = jnp.where(oh1, -jnp.inf, biased)
    m2 = jnp.max(b2, axis=1, keepdims=True)
    i2 = jnp.min(jnp.where(b2 >= m2, ii, big), axis=1, keepdims=True)
    oh2 = ii == i2
    return jnp.where(oh1 | oh2, scores, 0.0)


def _router_body(x_ref, wr_ref, bias_ref, w_ref):
    w_ref[...] = _router_weights(x_ref[...], wr_ref[...], bias_ref[...])


@jax.jit
def _router(x, wr, bias2d):
    return pl.pallas_call(
        _router_body,
        grid=(T // BT,),
        in_specs=[
            pl.BlockSpec((BT, D), lambda t: (t, 0)),
            pl.BlockSpec((NEXP, D), lambda t: (0, 0)),
            pl.BlockSpec((1, NEXP), lambda t: (0, 0)),
        ],
        out_specs=pl.BlockSpec((BT, NEXP), lambda t: (t, 0)),
        out_shape=jax.ShapeDtypeStruct((T, NEXP), jnp.float32),
    )(x, wr, bias2d)


def _ffn_body(xb_ref, w_ref, wg_ref, wu_ref, wd_ref, out_ref):
    e = pl.program_id(0)
    f = pl.program_id(1)
    x = xb_ref[...]
    w = w_ref[...]

    @pl.when((e == 0) & (f == 0))
    def _init():
        zero_w = jnp.sum(w[:, E:], axis=1, keepdims=True)
        out_ref[...] = zero_w * x.astype(jnp.float32)

    col = lax.broadcasted_iota(jnp.int32, w.shape, 1)
    we = jnp.sum(jnp.where(col == e, w, 0.0), axis=1, keepdims=True)
    swe = SCALE * we
    wg = wg_ref[0]
    wu = wu_ref[0]
    g = lax.dot_general(x, wg, (((1,), (1,)), ((), ())),
                        preferred_element_type=jnp.float32)
    u = lax.dot_general(x, wu, (((1,), (1,)), ((), ())),
                        preferred_element_type=jnp.float32)
    h = (g * jax.nn.sigmoid(g) * u).astype(jnp.bfloat16)
    for d in range(D // DC):
        wd = wd_ref[0, pl.ds(d * DC, DC), :]
        y = lax.dot_general(h, wd, (((1,), (1,)), ((), ())),
                            preferred_element_type=jnp.float32)
        out_ref[:, pl.ds(d * DC, DC)] += swe * y


@jax.jit
def _moe(x, wr, bias2d, wg, wu, wd):
    w_slots = _router(x, wr, bias2d)
    xb = x.astype(jnp.bfloat16)
    wgb = wg.astype(jnp.bfloat16)
    wub = wu.astype(jnp.bfloat16)
    wdb = wd.astype(jnp.bfloat16)
    return pl.pallas_call(
        _ffn_body,
        grid=(E, DFF // FC),
        in_specs=[
            pl.BlockSpec((T, D), lambda e, f: (0, 0)),
            pl.BlockSpec((T, NEXP), lambda e, f: (0, 0)),
            pl.BlockSpec((1, FC, D), lambda e, f: (e, f, 0)),
            pl.BlockSpec((1, FC, D), lambda e, f: (e, f, 0)),
            pl.BlockSpec((1, D, FC), lambda e, f: (e, 0, f)),
        ],
        out_specs=pl.BlockSpec((T, D), lambda e, f: (0, 0)),
        out_shape=jax.ShapeDtypeStruct((T, D), jnp.float32),
        compiler_params=pltpu.CompilerParams(
            dimension_semantics=("arbitrary", "arbitrary"),
        ),
    )(xb, w_slots, wgb, wub, wdb)


def kernel(hidden_states, W_router, correction_bias, W_gate, W_up, W_down):
    bias2d = correction_bias.reshape(1, NEXP)
    return _moe(hidden_states, W_router, bias2d, W_gate, W_up, W_down)
